# baseline (speedup 1.0000x reference)
import functools

import jax
import jax.numpy as jnp
from jax import lax
from jax.experimental import pallas as pl
from jax.experimental.pallas import tpu as pltpu

_BINS = 30
_SCALE = 30 - 0.0001
_LANES = 128
_BR = 256
_CH = 8
_LEVELS = 12


def _bins_of(x, t):
    g = jnp.abs(jax.nn.sigmoid(x) - t)
    return jnp.floor(g * _SCALE).astype(jnp.int32)


def _csa(a, b, c):
    u = a ^ b
    return u ^ c, (a & b) | (u & c)


def _fused_kernel(x_ref, t_ref, out_ref, planes_ref, l_ref, b_ref, beta_ref,
                  *, nblocks, tot):
    i = pl.program_id(0)

    @pl.when(i == 0)
    def _init():
        planes_ref[...] = jnp.zeros_like(planes_ref)

    @pl.when(i < nblocks)
    def _phase_a():
        x = x_ref[...]
        t = t_ref[...]
        bb = _bins_of(x, t)
        v = jnp.left_shift(jnp.int32(1), bb)

        vals = {0: [v[k * _CH:(k + 1) * _CH, :] for k in range(_BR // _CH)]}
        j = 0
        while j in vals:
            lv = vals[j]
            carries = []
            while len(lv) >= 3:
                s, co = _csa(lv.pop(), lv.pop(), lv.pop())
                lv.append(s)
                carries.append(co)
            if len(lv) == 2:
                a0, a1 = lv
                lv = [a0 ^ a1]
                carries.append(a0 & a1)
            if carries:
                vals[j + 1] = carries
            if lv:
                carry = lv[0]
                for lvl in range(j, _LEVELS):
                    old = planes_ref[lvl]
                    planes_ref[lvl] = old ^ carry
                    carry = old & carry
            j += 1

        lval = jnp.maximum(x, 0.0) - x * t + jnp.log1p(jnp.exp(-jnp.abs(x)))
        l_ref[pl.ds(i * _BR, _BR), :] = lval.astype(jnp.bfloat16)
        b_ref[pl.ds(i * _BR, _BR), :] = bb.astype(jnp.int8)

    @pl.when(i == nblocks)
    def _make_beta():
        li = lax.broadcasted_iota(jnp.int32, (1, _LANES), 1)
        vec = jnp.zeros((1, _LANES), jnp.float32)
        for k in range(_BINS):
            c = jnp.float32(0.0)
            for lvl in range(_LEVELS):
                bits = (planes_ref[lvl] >> k) & 1
                c = c + jnp.float32(1 << lvl) * jnp.sum(bits).astype(jnp.float32)
            vec = vec + jnp.where(li == k, c, 0.0)
        ne = jnp.sum(jnp.where((li < _BINS) & (vec > 0), 1.0, 0.0))
        beta_ref[...] = tot / jnp.clip(vec * ne, 0.0001, None)

    @pl.when(i >= nblocks)
    def _phase_b():
        j = i - nblocks
        lval = l_ref[pl.ds(j * _BR, _BR), :].astype(jnp.float32)
        bb = b_ref[pl.ds(j * _BR, _BR), :].astype(jnp.int32)
        tab = jnp.broadcast_to(beta_ref[...][:, :32], (_BR, 32))
        w = jnp.take_along_axis(tab, bb, axis=1)
        out_ref[...] = jnp.mean(w * lval, axis=1)


def kernel(logits, target):
    rows, cols = logits.shape
    nblocks = rows // _BR
    tot = float(logits.size)

    def in_idx(i):
        return (jnp.where(i < nblocks, i, nblocks - 1), 0)

    def out_idx(i):
        return (jnp.where(i >= nblocks, i - nblocks, 0),)

    return pl.pallas_call(
        functools.partial(_fused_kernel, nblocks=nblocks, tot=tot),
        grid=(2 * nblocks,),
        in_specs=[
            pl.BlockSpec((_BR, cols), in_idx),
            pl.BlockSpec((_BR, cols), in_idx),
        ],
        out_specs=pl.BlockSpec((_BR,), out_idx),
        out_shape=jax.ShapeDtypeStruct((rows,), jnp.float32),
        scratch_shapes=[
            pltpu.VMEM((_LEVELS, _CH, cols), jnp.int32),
            pltpu.VMEM((rows, cols), jnp.bfloat16),
            pltpu.VMEM((rows, cols), jnp.int8),
            pltpu.VMEM((1, _LANES), jnp.float32),
        ],
        compiler_params=pltpu.CompilerParams(
            dimension_semantics=("arbitrary",),
        ),
    )(logits, target)


# speedup vs baseline: 1.2758x; 1.2758x over previous
import functools

import jax
import jax.numpy as jnp
from jax import lax
from jax.experimental import pallas as pl
from jax.experimental.pallas import tpu as pltpu

_BINS = 30
_SCALE = 30 - 0.0001
_LANES = 128
_BR = 1024
_CH = 8
_LEVELS = 12


def _bins_of(x, t):
    g = jnp.abs(jax.nn.sigmoid(x) - t)
    return jnp.floor(g * _SCALE).astype(jnp.int32)


def _csa(a, b, c):
    u = a ^ b
    return u ^ c, (a & b) | (u & c)


def _hist_kernel(x_ref, t_ref, beta_ref, bidx_ref, planes_ref, *, nblocks, tot):
    i = pl.program_id(0)

    @pl.when(i == 0)
    def _init():
        planes_ref[...] = jnp.zeros_like(planes_ref)

    bb = _bins_of(x_ref[...], t_ref[...])
    bidx_ref[...] = bb.astype(jnp.int8)
    v = jnp.left_shift(jnp.int32(1), bb)

    vals = {0: [v[k * _CH:(k + 1) * _CH, :] for k in range(_BR // _CH)]}
    j = 0
    while j in vals:
        lv = vals[j]
        carries = []
        while len(lv) >= 3:
            s, co = _csa(lv.pop(), lv.pop(), lv.pop())
            lv.append(s)
            carries.append(co)
        if len(lv) == 2:
            a0, a1 = lv
            lv = [a0 ^ a1]
            carries.append(a0 & a1)
        if carries:
            vals[j + 1] = carries
        if lv:
            carry = lv[0]
            for lvl in range(j, _LEVELS):
                old = planes_ref[lvl]
                planes_ref[lvl] = old ^ carry
                carry = old & carry
        j += 1

    @pl.when(i == nblocks - 1)
    def _extract():
        li = lax.broadcasted_iota(jnp.int32, (1, _LANES), 1)
        vec = jnp.zeros((1, _LANES), jnp.float32)
        for k in range(_BINS):
            c = jnp.float32(0.0)
            for lvl in range(_LEVELS):
                bits = (planes_ref[lvl] >> k) & 1
                c = c + jnp.float32(1 << lvl) * jnp.sum(bits).astype(jnp.float32)
            vec = vec + jnp.where(li == k, c, 0.0)
        ne = jnp.sum(jnp.where((li < _BINS) & (vec > 0), 1.0, 0.0))
        beta_ref[...] = tot / jnp.clip(vec * ne, 0.0001, None)


def _loss_kernel(beta_ref, bidx_ref, x_ref, t_ref, out_ref):
    x = x_ref[...]
    t = t_ref[...]
    bb = bidx_ref[...].astype(jnp.int32)
    tab = jnp.broadcast_to(beta_ref[...][:, :32], (x.shape[0], 32))
    w = jnp.take_along_axis(tab, bb, axis=1)
    loss = w * (jnp.maximum(x, 0.0) - x * t + jnp.log1p(jnp.exp(-jnp.abs(x))))
    ones = jnp.full((x.shape[1], 1), 1.0 / x.shape[1], dtype=jnp.float32)
    out_ref[...] = lax.dot_general(
        loss, ones, (((1,), (0,)), ((), ())),
        preferred_element_type=jnp.float32,
    )[:, 0]


def kernel(logits, target):
    rows, cols = logits.shape
    nblocks = rows // _BR
    tot = float(logits.size)

    beta, bidx = pl.pallas_call(
        functools.partial(_hist_kernel, nblocks=nblocks, tot=tot),
        grid=(nblocks,),
        in_specs=[
            pl.BlockSpec((_BR, cols), lambda i: (i, 0)),
            pl.BlockSpec((_BR, cols), lambda i: (i, 0)),
        ],
        out_specs=[
            pl.BlockSpec((1, _LANES), lambda i: (0, 0)),
            pl.BlockSpec((_BR, cols), lambda i: (i, 0)),
        ],
        out_shape=[
            jax.ShapeDtypeStruct((1, _LANES), jnp.float32),
            jax.ShapeDtypeStruct((rows, cols), jnp.int8),
        ],
        scratch_shapes=[pltpu.VMEM((_LEVELS, _CH, cols), jnp.int32)],
        compiler_params=pltpu.CompilerParams(
            dimension_semantics=("arbitrary",),
        ),
    )(logits, target)

    out = pl.pallas_call(
        _loss_kernel,
        grid=(nblocks,),
        in_specs=[
            pl.BlockSpec((1, _LANES), lambda i: (0, 0)),
            pl.BlockSpec((_BR, cols), lambda i: (i, 0)),
            pl.BlockSpec((_BR, cols), lambda i: (i, 0)),
            pl.BlockSpec((_BR, cols), lambda i: (i, 0)),
        ],
        out_specs=pl.BlockSpec((_BR,), lambda i: (i,)),
        out_shape=jax.ShapeDtypeStruct((rows,), jnp.float32),
        compiler_params=pltpu.CompilerParams(
            dimension_semantics=("arbitrary",),
        ),
    )(beta, bidx, logits, target)
    return out
